# Initial kernel scaffold; baseline (speedup 1.0000x reference)
import jax
import jax.numpy as jnp
from jax.experimental import pallas as pl


def kernel(x, edges, W1, b1, W2, b2):
    raise NotImplementedError("write your pallas kernel here")


# speedup vs baseline: 18.9668x; 18.9668x over previous
import dataclasses
import functools

import jax
import jax.numpy as jnp
from jax import lax
from jax.experimental import pallas as pl
from jax.experimental.pallas import tpu as pltpu
from jax.experimental.pallas import tpu_sc as plsc

N = 10000
NPAD = 10240
E = 320000
F_IN = 128
F_HID = 156
F_OUT = 7
PAD_OUT = 16
CHUNK = 128
NCHUNK = E // CHUNK
NC = 2
NS = 16
NW = NC * NS
ITERS = (NCHUNK + NW - 1) // NW
RPS = NPAD // NS


def _sc_mesh():
    return plsc.VectorSubcoreMesh(core_axis_name="c", subcore_axis_name="s")


def _sc_params():
    cp = pltpu.CompilerParams()
    fields = pltpu.CompilerParams.__dataclass_fields__
    if "needs_layout_passes" in fields:
        cp = dataclasses.replace(cp, needs_layout_passes=False)
    if "use_tc_tiling_on_sc" in fields:
        cp = dataclasses.replace(cp, use_tc_tiling_on_sc=False)
    return cp


def _deg_sc(dst):
    @functools.partial(
        pl.kernel,
        out_type=jax.ShapeDtypeStruct((NC, NPAD, PAD_OUT), jnp.float32),
        mesh=_sc_mesh(),
        scratch_types=[
            pltpu.VMEM((CHUNK,), jnp.int32),
            pltpu.VMEM((CHUNK, PAD_OUT), jnp.float32),
            pltpu.VMEM_SHARED((NPAD, PAD_OUT), jnp.float32),
        ],
        compiler_params=_sc_params(),
    )
    def deg_kernel(dst_hbm, out_hbm, idx_v, ones_v, acc_sh):
        cid = lax.axis_index("c")
        sid = lax.axis_index("s")
        wid = sid * NC + cid

        @pl.loop(0, CHUNK)
        def _(i):
            ones_v[i, :] = jnp.zeros((PAD_OUT,), jnp.float32)

        for j in range(RPS // CHUNK):
            pltpu.sync_copy(
                ones_v,
                acc_sh.at[pl.ds(sid * RPS + j * CHUNK, CHUNK)],
            )

        @pl.loop(0, CHUNK)
        def _(i):
            ones_v[i, :] = jnp.full((PAD_OUT,), 1.0, jnp.float32)

        plsc.subcore_barrier()

        @pl.loop(0, ITERS)
        def _(i):
            c = wid + NW * i

            @pl.when(c < NCHUNK)
            def _():
                pltpu.sync_copy(dst_hbm.at[pl.ds(c * CHUNK, CHUNK)], idx_v)
                pltpu.sync_copy(ones_v, acc_sh.at[idx_v], add=True)

        plsc.subcore_barrier()
        pltpu.sync_copy(
            acc_sh.at[pl.ds(sid * RPS, RPS)],
            out_hbm.at[cid, pl.ds(sid * RPS, RPS)],
        )

    return deg_kernel(dst)


def _agg_sc(y, src, dst, d):
    @functools.partial(
        pl.kernel,
        out_type=jax.ShapeDtypeStruct((NC, NPAD, d), jnp.float32),
        mesh=_sc_mesh(),
        scratch_types=[
            pltpu.VMEM((CHUNK,), jnp.int32),
            pltpu.VMEM((CHUNK,), jnp.int32),
            pltpu.VMEM((CHUNK, d), jnp.float32),
            pltpu.VMEM_SHARED((NPAD, d), jnp.float32),
        ],
        compiler_params=_sc_params(),
    )
    def agg_kernel(y_hbm, src_hbm, dst_hbm, out_hbm, sidx_v, didx_v, rows_v, acc_sh):
        cid = lax.axis_index("c")
        sid = lax.axis_index("s")
        wid = sid * NC + cid

        @pl.loop(0, CHUNK)
        def _(i):
            @pl.loop(0, d // 16)
            def _(j):
                rows_v[i, pl.ds(j * 16, 16)] = jnp.zeros((16,), jnp.float32)

        for j in range(RPS // CHUNK):
            pltpu.sync_copy(
                rows_v,
                acc_sh.at[pl.ds(sid * RPS + j * CHUNK, CHUNK)],
            )

        plsc.subcore_barrier()

        @pl.loop(0, ITERS)
        def _(i):
            c = wid + NW * i

            @pl.when(c < NCHUNK)
            def _():
                pltpu.sync_copy(src_hbm.at[pl.ds(c * CHUNK, CHUNK)], sidx_v)
                pltpu.sync_copy(dst_hbm.at[pl.ds(c * CHUNK, CHUNK)], didx_v)
                pltpu.sync_copy(y_hbm.at[sidx_v], rows_v)
                pltpu.sync_copy(rows_v, acc_sh.at[didx_v], add=True)

        plsc.subcore_barrier()
        pltpu.sync_copy(
            acc_sh.at[pl.ds(sid * RPS, RPS)],
            out_hbm.at[cid, pl.ds(sid * RPS, RPS)],
        )

    return agg_kernel(y, src, dst)


def _dinv(d0, d1):
    deg = d0[:, 0:1] + d1[:, 0:1] + 1.0
    return lax.rsqrt(deg)


def _tc_prescale(deg0, deg1, x):
    def body(d0, d1, x_ref, y_ref):
        y_ref[...] = x_ref[...] * _dinv(d0, d1)

    return pl.pallas_call(
        body, out_shape=jax.ShapeDtypeStruct((N, F_IN), jnp.float32)
    )(deg0, deg1, x)


def _tc_mid(s0, s1, y1, deg0, deg1, W1, b1, W2p):
    def body(s0_r, s1_r, y1_r, d0, d1, w1_r, b1_r, w2_r, y2_r):
        dinv = _dinv(d0, d1)
        u = (s0_r[...] + s1_r[...] + y1_r[...]) * dinv
        h = jnp.dot(u, w1_r[...], preferred_element_type=jnp.float32) + b1_r[...]
        h = jnp.maximum(h, 0.0)
        z = jnp.dot(h, w2_r[...], preferred_element_type=jnp.float32)
        y2_r[...] = z * dinv

    return pl.pallas_call(
        body, out_shape=jax.ShapeDtypeStruct((N, PAD_OUT), jnp.float32)
    )(s0, s1, y1, deg0, deg1, W1, b1, W2p)


def _tc_post(s0, s1, y2, deg0, deg1, b2):
    def body(s0_r, s1_r, y2_r, d0, d1, b2_r, o_r):
        dinv = _dinv(d0, d1)
        v = (s0_r[...] + s1_r[...] + y2_r[...]) * dinv
        logits = v[:, 0:F_OUT] + b2_r[...]
        m = jnp.max(logits, axis=1, keepdims=True)
        sh = logits - m
        lse = jnp.log(jnp.sum(jnp.exp(sh), axis=1, keepdims=True))
        o_r[...] = sh - lse

    return pl.pallas_call(
        body, out_shape=jax.ShapeDtypeStruct((N, F_OUT), jnp.float32)
    )(s0, s1, y2, deg0, deg1, b2)


def kernel(x, edges, W1, b1, W2, b2):
    src = edges[0].astype(jnp.int32)
    dst = edges[1].astype(jnp.int32)
    W2p = jnp.zeros((F_HID, PAD_OUT), jnp.float32).at[:, :F_OUT].set(W2)
    b1r = b1.reshape(1, F_HID)
    b2r = b2.reshape(1, F_OUT)

    deg = _deg_sc(dst)
    deg0, deg1 = deg[0, :N], deg[1, :N]
    y1 = _tc_prescale(deg0, deg1, x)
    S1 = _agg_sc(y1, src, dst, F_IN)
    y2 = _tc_mid(S1[0, :N], S1[1, :N], y1, deg0, deg1, W1, b1r, W2p)
    S2 = _agg_sc(y2, src, dst, PAD_OUT)
    return _tc_post(S2[0, :N], S2[1, :N], y2, deg0, deg1, b2r)


# speedup vs baseline: 32.5240x; 1.7148x over previous
import dataclasses
import functools

import jax
import jax.numpy as jnp
from jax import lax
from jax.experimental import pallas as pl
from jax.experimental.pallas import tpu as pltpu
from jax.experimental.pallas import tpu_sc as plsc

N = 10000
NPAD = 10240
E = 320000
F_IN = 128
F_HID = 156
F_OUT = 7
PAD_OUT = 16
CHUNK = 128
NCHUNK = E // CHUNK
NC = 2
NS = 16
NW = NC * NS
ITERS = (NCHUNK + NW - 1) // NW
RPS = NPAD // NS


def _sc_mesh():
    return plsc.VectorSubcoreMesh(core_axis_name="c", subcore_axis_name="s")


def _sc_params():
    cp = pltpu.CompilerParams()
    fields = pltpu.CompilerParams.__dataclass_fields__
    if "needs_layout_passes" in fields:
        cp = dataclasses.replace(cp, needs_layout_passes=False)
    if "use_tc_tiling_on_sc" in fields:
        cp = dataclasses.replace(cp, use_tc_tiling_on_sc=False)
    return cp


def _zero_fill(ref, nrows, d):
    @pl.loop(0, nrows)
    def _(i):
        @pl.loop(0, d // 16)
        def _(j):
            ref[i, pl.ds(j * 16, 16)] = jnp.zeros((16,), jnp.float32)


def _deg_sc(dst):
    @functools.partial(
        pl.kernel,
        out_type=jax.ShapeDtypeStruct((NC, NPAD, PAD_OUT), jnp.float32),
        mesh=_sc_mesh(),
        scratch_types=[
            pltpu.VMEM((CHUNK,), jnp.int32),
            pltpu.VMEM((CHUNK,), jnp.int32),
            pltpu.VMEM((CHUNK,), jnp.int32),
            pltpu.VMEM((CHUNK,), jnp.int32),
            pltpu.VMEM((CHUNK, PAD_OUT), jnp.float32),
            pltpu.SemaphoreType.DMA,
            pltpu.SemaphoreType.DMA,
            pltpu.SemaphoreType.DMA,
            pltpu.SemaphoreType.DMA,
            pltpu.SemaphoreType.DMA,
            pltpu.SemaphoreType.DMA,
            pltpu.VMEM_SHARED((NPAD, PAD_OUT), jnp.float32),
        ],
        compiler_params=_sc_params(),
    )
    def deg_kernel(dst_hbm, out_hbm, di0, di1, di2, di3, ones_v,
                   semi0, semi1, semi2, semi3, sems0, sems1, acc_sh):
        cid = lax.axis_index("c")
        sid = lax.axis_index("s")
        wid = sid * NC + cid
        didx = [di0, di1, di2, di3]
        semi = [semi0, semi1, semi2, semi3]
        sems = [sems0, sems1]

        _zero_fill(ones_v, CHUNK, PAD_OUT)
        for j in range(RPS // CHUNK):
            pltpu.sync_copy(ones_v, acc_sh.at[pl.ds(sid * RPS + j * CHUNK, CHUNK)])

        @pl.loop(0, CHUNK)
        def _(i):
            ones_v[i, :] = jnp.full((PAD_OUT,), 1.0, jnp.float32)

        plsc.subcore_barrier()

        def idx_start(chunk, slot):
            c = wid + NW * chunk
            pltpu.async_copy(dst_hbm.at[pl.ds(c * CHUNK, CHUNK)], didx[slot],
                             semi[slot])

        def idx_wait(slot):
            pltpu.make_async_copy(dst_hbm.at[pl.ds(0, CHUNK)], didx[slot],
                                  semi[slot]).wait()

        def scat_start(islot, ss):
            pltpu.async_copy(ones_v, acc_sh.at[didx[islot]], sems[ss], add=True)

        def scat_wait(ss):
            pltpu.make_async_copy(ones_v, acc_sh.at[di0], sems[ss]).wait()

        idx_start(0, 0)
        idx_start(1, 1)
        idx_start(2, 2); idx_wait(0); scat_start(0, 0)
        idx_start(3, 3); idx_wait(1); scat_start(1, 1)
        scat_wait(0); idx_start(4, 0); idx_wait(2); scat_start(2, 0)
        scat_wait(1); idx_start(5, 1); idx_wait(3); scat_start(3, 1)

        @pl.loop(4, 76, step=4)
        def _(o):
            for k in range(4):
                scat_wait(k % 2)
                idx_start(o + k + 2, (k + 2) % 4)
                idx_wait(k)
                scat_start(k, k % 2)

        scat_wait(0)

        @pl.when(wid < 4)
        def _():
            idx_start(78, 2)

        idx_wait(0); scat_start(0, 0)
        scat_wait(1); idx_wait(1); scat_start(1, 1)
        scat_wait(0)

        @pl.when(wid < 4)
        def _():
            idx_wait(2); scat_start(2, 0)

        scat_wait(1)

        @pl.when(wid < 4)
        def _():
            scat_wait(0)

        plsc.subcore_barrier()
        pltpu.sync_copy(
            acc_sh.at[pl.ds(sid * RPS, RPS)],
            out_hbm.at[cid, pl.ds(sid * RPS, RPS)],
        )

    return deg_kernel(dst)


def _agg_sc(y, src, dst, d):
    @functools.partial(
        pl.kernel,
        out_type=jax.ShapeDtypeStruct((NC, NPAD, d), jnp.float32),
        mesh=_sc_mesh(),
        scratch_types=[
            pltpu.VMEM((CHUNK,), jnp.int32),
            pltpu.VMEM((CHUNK,), jnp.int32),
            pltpu.VMEM((CHUNK,), jnp.int32),
            pltpu.VMEM((CHUNK,), jnp.int32),
            pltpu.VMEM((CHUNK,), jnp.int32),
            pltpu.VMEM((CHUNK,), jnp.int32),
            pltpu.VMEM((CHUNK,), jnp.int32),
            pltpu.VMEM((CHUNK,), jnp.int32),
            pltpu.VMEM((CHUNK, d), jnp.float32),
            pltpu.VMEM((CHUNK, d), jnp.float32),
            pltpu.SemaphoreType.DMA,
            pltpu.SemaphoreType.DMA,
            pltpu.SemaphoreType.DMA,
            pltpu.SemaphoreType.DMA,
            pltpu.SemaphoreType.DMA,
            pltpu.SemaphoreType.DMA,
            pltpu.VMEM_SHARED((NPAD, d), jnp.float32),
        ],
        compiler_params=_sc_params(),
    )
    def agg_kernel(y_hbm, src_hbm, dst_hbm, out_hbm,
                   si0, si1, si2, si3, di0, di1, di2, di3,
                   rows0, rows1,
                   semi0, semi1, semi2, semi3, sems0, sems1, acc_sh):
        cid = lax.axis_index("c")
        sid = lax.axis_index("s")
        wid = sid * NC + cid
        sidx = [si0, si1, si2, si3]
        didx = [di0, di1, di2, di3]
        rows = [rows0, rows1]
        semi = [semi0, semi1, semi2, semi3]
        sems = [sems0, sems1]

        _zero_fill(rows0, CHUNK, d)
        for j in range(RPS // CHUNK):
            pltpu.sync_copy(rows0, acc_sh.at[pl.ds(sid * RPS + j * CHUNK, CHUNK)])

        plsc.subcore_barrier()

        def idx_start(chunk, slot):
            c = wid + NW * chunk
            pltpu.async_copy(src_hbm.at[pl.ds(c * CHUNK, CHUNK)], sidx[slot],
                             semi[slot])
            pltpu.async_copy(dst_hbm.at[pl.ds(c * CHUNK, CHUNK)], didx[slot],
                             semi[slot])

        def idx_wait(slot):
            pltpu.make_async_copy(src_hbm.at[pl.ds(0, CHUNK)], sidx[slot],
                                  semi[slot]).wait()
            pltpu.make_async_copy(dst_hbm.at[pl.ds(0, CHUNK)], didx[slot],
                                  semi[slot]).wait()

        def gather(islot, rs):
            pltpu.sync_copy(y_hbm.at[sidx[islot]], rows[rs])

        def scat_start(islot, rs):
            pltpu.async_copy(rows[rs], acc_sh.at[didx[islot]], sems[rs], add=True)

        def scat_wait(rs):
            pltpu.make_async_copy(rows[rs], acc_sh.at[di0], sems[rs]).wait()

        idx_start(0, 0)
        idx_start(1, 1)
        idx_start(2, 2); idx_wait(0); gather(0, 0); scat_start(0, 0)
        idx_start(3, 3); idx_wait(1); gather(1, 1); scat_start(1, 1)
        scat_wait(0); idx_start(4, 0); idx_wait(2); gather(2, 0); scat_start(2, 0)
        scat_wait(1); idx_start(5, 1); idx_wait(3); gather(3, 1); scat_start(3, 1)

        @pl.loop(4, 76, step=4)
        def _(o):
            for k in range(4):
                scat_wait(k % 2)
                idx_start(o + k + 2, (k + 2) % 4)
                idx_wait(k)
                gather(k, k % 2)
                scat_start(k, k % 2)

        scat_wait(0)

        @pl.when(wid < 4)
        def _():
            idx_start(78, 2)

        idx_wait(0); gather(0, 0); scat_start(0, 0)
        scat_wait(1); idx_wait(1); gather(1, 1); scat_start(1, 1)
        scat_wait(0)

        @pl.when(wid < 4)
        def _():
            idx_wait(2); gather(2, 0); scat_start(2, 0)

        scat_wait(1)

        @pl.when(wid < 4)
        def _():
            scat_wait(0)

        plsc.subcore_barrier()
        pltpu.sync_copy(
            acc_sh.at[pl.ds(sid * RPS, RPS)],
            out_hbm.at[cid, pl.ds(sid * RPS, RPS)],
        )

    return agg_kernel(y, src, dst)


def _dinv(d0, d1):
    deg = d0[:, 0:1] + d1[:, 0:1] + 1.0
    return lax.rsqrt(deg)


def _tc_prescale(deg0, deg1, x):
    def body(d0, d1, x_ref, y_ref):
        y_ref[...] = x_ref[...] * _dinv(d0, d1)

    return pl.pallas_call(
        body, out_shape=jax.ShapeDtypeStruct((N, F_IN), jnp.float32)
    )(deg0, deg1, x)


def _tc_mid(s0, s1, y1, deg0, deg1, W1, b1, W2p):
    def body(s0_r, s1_r, y1_r, d0, d1, w1_r, b1_r, w2_r, y2_r):
        dinv = _dinv(d0, d1)
        u = (s0_r[...] + s1_r[...] + y1_r[...]) * dinv
        h = jnp.dot(u, w1_r[...], preferred_element_type=jnp.float32) + b1_r[...]
        h = jnp.maximum(h, 0.0)
        z = jnp.dot(h, w2_r[...], preferred_element_type=jnp.float32)
        y2_r[...] = z * dinv

    return pl.pallas_call(
        body, out_shape=jax.ShapeDtypeStruct((N, PAD_OUT), jnp.float32)
    )(s0, s1, y1, deg0, deg1, W1, b1, W2p)


def _tc_post(s0, s1, y2, deg0, deg1, b2):
    def body(s0_r, s1_r, y2_r, d0, d1, b2_r, o_r):
        dinv = _dinv(d0, d1)
        v = (s0_r[...] + s1_r[...] + y2_r[...]) * dinv
        logits = v[:, 0:F_OUT] + b2_r[...]
        m = jnp.max(logits, axis=1, keepdims=True)
        sh = logits - m
        lse = jnp.log(jnp.sum(jnp.exp(sh), axis=1, keepdims=True))
        o_r[...] = sh - lse

    return pl.pallas_call(
        body, out_shape=jax.ShapeDtypeStruct((N, F_OUT), jnp.float32)
    )(s0, s1, y2, deg0, deg1, b2)


def kernel(x, edges, W1, b1, W2, b2):
    src = edges[0].astype(jnp.int32)
    dst = edges[1].astype(jnp.int32)
    W2p = jnp.zeros((F_HID, PAD_OUT), jnp.float32).at[:, :F_OUT].set(W2)
    b1r = b1.reshape(1, F_HID)
    b2r = b2.reshape(1, F_OUT)

    deg = _deg_sc(dst)
    deg0, deg1 = deg[0, :N], deg[1, :N]
    y1 = _tc_prescale(deg0, deg1, x)
    S1 = _agg_sc(y1, src, dst, F_IN)
    y2 = _tc_mid(S1[0, :N], S1[1, :N], y1, deg0, deg1, W1, b1r, W2p)
    S2 = _agg_sc(y2, src, dst, PAD_OUT)
    return _tc_post(S2[0, :N], S2[1, :N], y2, deg0, deg1, b2r)


# speedup vs baseline: 33.6362x; 1.0342x over previous
import dataclasses
import functools

import jax
import jax.numpy as jnp
from jax import lax
from jax.experimental import pallas as pl
from jax.experimental.pallas import tpu as pltpu
from jax.experimental.pallas import tpu_sc as plsc

N = 10000
NPAD = 10240
E = 320000
F_IN = 128
F_HID = 156
F_OUT = 7
PAD_OUT = 16
NC = 2
NS = 16
NW = NC * NS
RPS = NPAD // NS


def _sc_mesh():
    return plsc.VectorSubcoreMesh(core_axis_name="c", subcore_axis_name="s")


def _sc_params():
    cp = pltpu.CompilerParams()
    fields = pltpu.CompilerParams.__dataclass_fields__
    if "needs_layout_passes" in fields:
        cp = dataclasses.replace(cp, needs_layout_passes=False)
    if "use_tc_tiling_on_sc" in fields:
        cp = dataclasses.replace(cp, use_tc_tiling_on_sc=False)
    return cp


def _zero_fill(ref, nrows, d):
    @pl.loop(0, nrows)
    def _(i):
        @pl.loop(0, d // 16)
        def _(j):
            ref[i, pl.ds(j * 16, 16)] = jnp.zeros((16,), jnp.float32)


def _load_my_idx(idx2d_hbm, idx_v, wid, base_ch, extra_w):
    row0 = base_ch * wid + jnp.minimum(wid, extra_w)
    pltpu.sync_copy(idx2d_hbm.at[pl.ds(row0, base_ch)],
                    idx_v.at[pl.ds(0, base_ch)])

    @pl.when(wid < extra_w)
    def _():
        pltpu.sync_copy(idx2d_hbm.at[pl.ds(row0 + base_ch, 1)],
                        idx_v.at[pl.ds(base_ch, 1)])


def _deg_sc(dst2d):
    chunk = 128
    nchunk = E // chunk
    base_ch = nchunk // NW
    extra_w = nchunk - base_ch * NW
    npair = base_ch // 2

    @functools.partial(
        pl.kernel,
        out_type=jax.ShapeDtypeStruct((NC, NPAD, PAD_OUT), jnp.float32),
        mesh=_sc_mesh(),
        scratch_types=[
            pltpu.VMEM((base_ch + 1, chunk), jnp.int32),
            pltpu.VMEM((chunk, PAD_OUT), jnp.float32),
            pltpu.SemaphoreType.DMA,
            pltpu.SemaphoreType.DMA,
            pltpu.VMEM_SHARED((NPAD, PAD_OUT), jnp.float32),
        ],
        compiler_params=_sc_params(),
    )
    def deg_kernel(dst_hbm, out_hbm, didx_v, ones_v, sems0, sems1, acc_sh):
        cid = lax.axis_index("c")
        sid = lax.axis_index("s")
        wid = sid * NC + cid
        sems = [sems0, sems1]

        _zero_fill(ones_v, chunk, PAD_OUT)
        for j in range(RPS // chunk):
            pltpu.sync_copy(ones_v, acc_sh.at[pl.ds(sid * RPS + j * chunk, chunk)])

        @pl.loop(0, chunk)
        def _(i):
            ones_v[i, :] = jnp.full((PAD_OUT,), 1.0, jnp.float32)

        _load_my_idx(dst_hbm, didx_v, wid, base_ch, extra_w)
        plsc.subcore_barrier()

        def scat_start(j, ss):
            pltpu.async_copy(ones_v, acc_sh.at[didx_v.at[j]], sems[ss], add=True)

        def scat_wait(ss):
            pltpu.make_async_copy(ones_v, acc_sh.at[didx_v.at[0]],
                                  sems[ss]).wait()

        scat_start(0, 0)
        scat_start(1, 1)

        @pl.loop(1, npair)
        def _(p):
            scat_wait(0)
            scat_start(2 * p, 0)
            scat_wait(1)
            scat_start(2 * p + 1, 1)

        scat_wait(0)

        @pl.when(wid < extra_w)
        def _():
            scat_start(base_ch, 0)

        scat_wait(1)

        @pl.when(wid < extra_w)
        def _():
            scat_wait(0)

        plsc.subcore_barrier()
        pltpu.sync_copy(
            acc_sh.at[pl.ds(sid * RPS, RPS)],
            out_hbm.at[cid, pl.ds(sid * RPS, RPS)],
        )

    return deg_kernel(dst2d)


def _agg_sc(y, src2d, dst2d, d, chunk):
    nchunk = E // chunk
    base_ch = nchunk // NW
    extra_w = nchunk - base_ch * NW
    npair = base_ch // 2

    @functools.partial(
        pl.kernel,
        out_type=jax.ShapeDtypeStruct((NC, NPAD, d), jnp.float32),
        mesh=_sc_mesh(),
        scratch_types=[
            pltpu.VMEM((base_ch + 1, chunk), jnp.int32),
            pltpu.VMEM((base_ch + 1, chunk), jnp.int32),
            pltpu.VMEM((chunk, d), jnp.float32),
            pltpu.VMEM((chunk, d), jnp.float32),
            pltpu.SemaphoreType.DMA,
            pltpu.SemaphoreType.DMA,
            pltpu.SemaphoreType.DMA,
            pltpu.SemaphoreType.DMA,
            pltpu.VMEM_SHARED((NPAD, d), jnp.float32),
        ],
        compiler_params=_sc_params(),
    )
    def agg_kernel(y_hbm, src_hbm, dst_hbm, out_hbm,
                   sidx_v, didx_v, rows0, rows1,
                   semg0, semg1, sems0, sems1, acc_sh):
        cid = lax.axis_index("c")
        sid = lax.axis_index("s")
        wid = sid * NC + cid
        rows = [rows0, rows1]
        semg = [semg0, semg1]
        sems = [sems0, sems1]

        _zero_fill(rows0, chunk, d)
        for j in range(RPS // chunk):
            pltpu.sync_copy(rows0, acc_sh.at[pl.ds(sid * RPS + j * chunk, chunk)])

        _load_my_idx(src_hbm, sidx_v, wid, base_ch, extra_w)
        _load_my_idx(dst_hbm, didx_v, wid, base_ch, extra_w)
        plsc.subcore_barrier()

        def gath_start(j, rs):
            pltpu.async_copy(y_hbm.at[sidx_v.at[j]], rows[rs], semg[rs])

        def gath_wait(rs):
            pltpu.make_async_copy(y_hbm.at[sidx_v.at[0]], rows[rs],
                                  semg[rs]).wait()

        def scat_start(j, rs):
            pltpu.async_copy(rows[rs], acc_sh.at[didx_v.at[j]], sems[rs],
                             add=True)

        def scat_wait(rs):
            pltpu.make_async_copy(rows[0], acc_sh.at[didx_v.at[0]],
                                  sems[rs]).wait()

        gath_start(0, 0)
        gath_start(1, 1)
        gath_wait(0)
        scat_start(0, 0)
        gath_wait(1)
        scat_start(1, 1)

        @pl.loop(1, npair)
        def _(p):
            scat_wait(0)
            gath_start(2 * p, 0)
            scat_wait(1)
            gath_start(2 * p + 1, 1)
            gath_wait(0)
            scat_start(2 * p, 0)
            gath_wait(1)
            scat_start(2 * p + 1, 1)

        scat_wait(0)

        @pl.when(wid < extra_w)
        def _():
            gath_start(base_ch, 0)
            gath_wait(0)
            scat_start(base_ch, 0)

        scat_wait(1)

        @pl.when(wid < extra_w)
        def _():
            scat_wait(0)

        plsc.subcore_barrier()
        pltpu.sync_copy(
            acc_sh.at[pl.ds(sid * RPS, RPS)],
            out_hbm.at[cid, pl.ds(sid * RPS, RPS)],
        )

    return agg_kernel(y, src2d, dst2d)


def _dinv_from(deg_r):
    deg = deg_r[0, 0:N, 0:1] + deg_r[1, 0:N, 0:1] + 1.0
    return lax.rsqrt(deg)


def _tc_prescale(deg, x):
    def body(d_r, x_ref, y_ref):
        y_ref[...] = x_ref[...] * _dinv_from(d_r)

    return pl.pallas_call(
        body, out_shape=jax.ShapeDtypeStruct((N, F_IN), jnp.float32)
    )(deg, x)


def _tc_mid(S1, y1, deg, W1, b1, W2p):
    def body(s_r, y1_r, d_r, w1_r, b1_r, w2_r, y2_r):
        dinv = _dinv_from(d_r)
        u = (s_r[0, 0:N, :] + s_r[1, 0:N, :] + y1_r[...]) * dinv
        h = jnp.dot(u, w1_r[...], preferred_element_type=jnp.float32) + b1_r[...]
        h = jnp.maximum(h, 0.0)
        z = jnp.dot(h, w2_r[...], preferred_element_type=jnp.float32)
        y2_r[...] = z * dinv

    return pl.pallas_call(
        body, out_shape=jax.ShapeDtypeStruct((N, PAD_OUT), jnp.float32)
    )(S1, y1, deg, W1, b1, W2p)


def _tc_post(S2, y2, deg, b2):
    def body(s_r, y2_r, d_r, b2_r, o_r):
        dinv = _dinv_from(d_r)
        v = (s_r[0, 0:N, :] + s_r[1, 0:N, :] + y2_r[...]) * dinv
        logits = v[:, 0:F_OUT] + b2_r[...]
        m = jnp.max(logits, axis=1, keepdims=True)
        sh = logits - m
        lse = jnp.log(jnp.sum(jnp.exp(sh), axis=1, keepdims=True))
        o_r[...] = sh - lse

    return pl.pallas_call(
        body, out_shape=jax.ShapeDtypeStruct((N, F_OUT), jnp.float32)
    )(S2, y2, deg, b2)


def kernel(x, edges, W1, b1, W2, b2):
    src = edges[0].astype(jnp.int32)
    dst = edges[1].astype(jnp.int32)
    W2p = jnp.zeros((F_HID, PAD_OUT), jnp.float32).at[:, :F_OUT].set(W2)
    b1r = b1.reshape(1, F_HID)
    b2r = b2.reshape(1, F_OUT)

    deg = _deg_sc(dst.reshape(E // 128, 128))
    y1 = _tc_prescale(deg, x)
    S1 = _agg_sc(y1, src.reshape(E // 64, 64), dst.reshape(E // 64, 64),
                 F_IN, 64)
    y2 = _tc_mid(S1, y1, deg, W1, b1r, W2p)
    S2 = _agg_sc(y2, src.reshape(E // 128, 128), dst.reshape(E // 128, 128),
                 PAD_OUT, 128)
    return _tc_post(S2, y2, deg, b2r)


# speedup vs baseline: 35.4735x; 1.0546x over previous
import dataclasses
import functools

import jax
import jax.numpy as jnp
from jax import lax
from jax.experimental import pallas as pl
from jax.experimental.pallas import tpu as pltpu
from jax.experimental.pallas import tpu_sc as plsc

N = 10000
NPAD = 10240
E = 320000
F_IN = 128
F_HID = 156
F_OUT = 7
PAD_OUT = 16
NC = 2
NS = 16
NW = NC * NS
RPS = NPAD // NS


def _sc_mesh():
    return plsc.VectorSubcoreMesh(core_axis_name="c", subcore_axis_name="s")


def _sc_params():
    cp = pltpu.CompilerParams()
    fields = pltpu.CompilerParams.__dataclass_fields__
    if "needs_layout_passes" in fields:
        cp = dataclasses.replace(cp, needs_layout_passes=False)
    if "use_tc_tiling_on_sc" in fields:
        cp = dataclasses.replace(cp, use_tc_tiling_on_sc=False)
    return cp


def _zero_fill(ref, nrows, d):
    @pl.loop(0, nrows)
    def _(i):
        @pl.loop(0, d // 16)
        def _(j):
            ref[i, pl.ds(j * 16, 16)] = jnp.zeros((16,), jnp.float32)


def _load_my_idx(idx2d_hbm, idx_v, wid, base_ch, extra_w):
    row0 = base_ch * wid + jnp.minimum(wid, extra_w)
    pltpu.sync_copy(idx2d_hbm.at[pl.ds(row0, base_ch)],
                    idx_v.at[pl.ds(0, base_ch)])

    @pl.when(wid < extra_w)
    def _():
        pltpu.sync_copy(idx2d_hbm.at[pl.ds(row0 + base_ch, 1)],
                        idx_v.at[pl.ds(base_ch, 1)])


def _deg_sc(dst2d):
    chunk = 128
    nchunk = E // chunk
    base_ch = nchunk // NW
    extra_w = nchunk - base_ch * NW
    npair = base_ch // 2

    @functools.partial(
        pl.kernel,
        out_type=jax.ShapeDtypeStruct((NC, NPAD, PAD_OUT), jnp.float32),
        mesh=_sc_mesh(),
        scratch_types=[
            pltpu.VMEM((base_ch + 1, chunk), jnp.int32),
            pltpu.VMEM((chunk, PAD_OUT), jnp.float32),
            pltpu.SemaphoreType.DMA,
            pltpu.SemaphoreType.DMA,
            pltpu.VMEM_SHARED((NPAD, PAD_OUT), jnp.float32),
        ],
        compiler_params=_sc_params(),
    )
    def deg_kernel(dst_hbm, out_hbm, didx_v, ones_v, sems0, sems1, acc_sh):
        cid = lax.axis_index("c")
        sid = lax.axis_index("s")
        wid = sid * NC + cid
        sems = [sems0, sems1]

        _zero_fill(ones_v, chunk, PAD_OUT)
        for j in range(RPS // chunk):
            pltpu.sync_copy(ones_v, acc_sh.at[pl.ds(sid * RPS + j * chunk, chunk)])

        @pl.loop(0, chunk)
        def _(i):
            ones_v[i, :] = jnp.full((PAD_OUT,), 1.0, jnp.float32)

        _load_my_idx(dst_hbm, didx_v, wid, base_ch, extra_w)
        plsc.subcore_barrier()

        def scat_start(j, ss):
            pltpu.async_copy(ones_v, acc_sh.at[didx_v.at[j]], sems[ss], add=True)

        def scat_wait(ss):
            pltpu.make_async_copy(ones_v, acc_sh.at[didx_v.at[0]],
                                  sems[ss]).wait()

        scat_start(0, 0)
        scat_start(1, 1)

        @pl.loop(1, npair)
        def _(p):
            scat_wait(0)
            scat_start(2 * p, 0)
            scat_wait(1)
            scat_start(2 * p + 1, 1)

        scat_wait(0)

        @pl.when(wid < extra_w)
        def _():
            scat_start(base_ch, 0)

        scat_wait(1)

        @pl.when(wid < extra_w)
        def _():
            scat_wait(0)

        plsc.subcore_barrier()
        pltpu.sync_copy(
            acc_sh.at[pl.ds(sid * RPS, RPS)],
            out_hbm.at[cid, pl.ds(sid * RPS, RPS)],
        )

    return deg_kernel(dst2d)


def _agg_sc(y, src2d, dst2d, d, chunk, stage):
    nchunk = E // chunk
    base_ch = nchunk // NW
    extra_w = nchunk - base_ch * NW
    npair = base_ch // 2

    scratch = [
        pltpu.VMEM((base_ch + 1, chunk), jnp.int32),
        pltpu.VMEM((base_ch + 1, chunk), jnp.int32),
        pltpu.VMEM((chunk, d), jnp.float32),
        pltpu.VMEM((chunk, d), jnp.float32),
        pltpu.SemaphoreType.DMA,
        pltpu.SemaphoreType.DMA,
        pltpu.SemaphoreType.DMA,
        pltpu.SemaphoreType.DMA,
        pltpu.VMEM_SHARED((NPAD, d), jnp.float32),
    ]
    if stage:
        scratch.append(pltpu.VMEM_SHARED((NPAD, d), jnp.float32))

    @functools.partial(
        pl.kernel,
        out_type=jax.ShapeDtypeStruct((NC, NPAD, d), jnp.float32),
        mesh=_sc_mesh(),
        scratch_types=scratch,
        compiler_params=_sc_params(),
    )
    def agg_kernel(y_hbm, src_hbm, dst_hbm, out_hbm,
                   sidx_v, didx_v, rows0, rows1,
                   semg0, semg1, sems0, sems1, acc_sh, *maybe_ysp):
        cid = lax.axis_index("c")
        sid = lax.axis_index("s")
        wid = sid * NC + cid
        rows = [rows0, rows1]
        semg = [semg0, semg1]
        sems = [sems0, sems1]
        y_src = maybe_ysp[0] if stage else y_hbm

        _zero_fill(rows0, chunk, d)
        for j in range(RPS // chunk):
            pltpu.sync_copy(rows0, acc_sh.at[pl.ds(sid * RPS + j * chunk, chunk)])

        if stage:
            pltpu.sync_copy(y_hbm.at[pl.ds(sid * RPS, RPS)],
                            maybe_ysp[0].at[pl.ds(sid * RPS, RPS)])

        _load_my_idx(src_hbm, sidx_v, wid, base_ch, extra_w)
        _load_my_idx(dst_hbm, didx_v, wid, base_ch, extra_w)
        plsc.subcore_barrier()

        def gath_start(j, rs):
            pltpu.async_copy(y_src.at[sidx_v.at[j]], rows[rs], semg[rs])

        def gath_wait(rs):
            pltpu.make_async_copy(y_src.at[sidx_v.at[0]], rows[rs],
                                  semg[rs]).wait()

        def scat_start(j, rs):
            pltpu.async_copy(rows[rs], acc_sh.at[didx_v.at[j]], sems[rs],
                             add=True)

        def scat_wait(rs):
            pltpu.make_async_copy(rows[0], acc_sh.at[didx_v.at[0]],
                                  sems[rs]).wait()

        gath_start(0, 0)
        gath_start(1, 1)
        gath_wait(0)
        scat_start(0, 0)
        gath_wait(1)
        scat_start(1, 1)

        @pl.loop(1, npair)
        def _(p):
            scat_wait(0)
            gath_start(2 * p, 0)
            scat_wait(1)
            gath_start(2 * p + 1, 1)
            gath_wait(0)
            scat_start(2 * p, 0)
            gath_wait(1)
            scat_start(2 * p + 1, 1)

        scat_wait(0)

        @pl.when(wid < extra_w)
        def _():
            gath_start(base_ch, 0)
            gath_wait(0)
            scat_start(base_ch, 0)

        scat_wait(1)

        @pl.when(wid < extra_w)
        def _():
            scat_wait(0)

        plsc.subcore_barrier()
        pltpu.sync_copy(
            acc_sh.at[pl.ds(sid * RPS, RPS)],
            out_hbm.at[cid, pl.ds(sid * RPS, RPS)],
        )

    return agg_kernel(y, src2d, dst2d)


def _dinv_from(deg_r):
    deg = deg_r[0, :, 0:1] + deg_r[1, :, 0:1] + 1.0
    return lax.rsqrt(deg)


def _tc_prescale(deg, x):
    def body(d_r, x_ref, y_ref):
        y_ref[0:N, :] = x_ref[...] * _dinv_from(d_r)[0:N]
        y_ref[N:NPAD, :] = jnp.zeros((NPAD - N, F_IN), jnp.float32)

    return pl.pallas_call(
        body, out_shape=jax.ShapeDtypeStruct((NPAD, F_IN), jnp.float32)
    )(deg, x)


def _tc_mid(S1, y1, deg, W1, b1, W2p):
    def body(s_r, y1_r, d_r, w1_r, b1_r, w2_r, y2_r):
        dinv = _dinv_from(d_r)
        u = (s_r[0] + s_r[1] + y1_r[...]) * dinv
        h = jnp.dot(u, w1_r[...], preferred_element_type=jnp.float32) + b1_r[...]
        h = jnp.maximum(h, 0.0)
        z = jnp.dot(h, w2_r[...], preferred_element_type=jnp.float32)
        y2_r[...] = z * dinv

    return pl.pallas_call(
        body, out_shape=jax.ShapeDtypeStruct((NPAD, PAD_OUT), jnp.float32)
    )(S1, y1, deg, W1, b1, W2p)


def _tc_post(S2, y2, deg, b2):
    def body(s_r, y2_r, d_r, b2_r, o_r):
        dinv = _dinv_from(d_r)[0:N]
        v = (s_r[0, 0:N, :] + s_r[1, 0:N, :] + y2_r[0:N, :]) * dinv
        logits = v[:, 0:F_OUT] + b2_r[...]
        m = jnp.max(logits, axis=1, keepdims=True)
        sh = logits - m
        lse = jnp.log(jnp.sum(jnp.exp(sh), axis=1, keepdims=True))
        o_r[...] = sh - lse

    return pl.pallas_call(
        body, out_shape=jax.ShapeDtypeStruct((N, F_OUT), jnp.float32)
    )(S2, y2, deg, b2)


def kernel(x, edges, W1, b1, W2, b2):
    src = edges[0].astype(jnp.int32)
    dst = edges[1].astype(jnp.int32)
    W2p = jnp.zeros((F_HID, PAD_OUT), jnp.float32).at[:, :F_OUT].set(W2)
    b1r = b1.reshape(1, F_HID)
    b2r = b2.reshape(1, F_OUT)

    deg = _deg_sc(dst.reshape(E // 128, 128))
    y1 = _tc_prescale(deg, x)
    S1 = _agg_sc(y1, src.reshape(E // 64, 64), dst.reshape(E // 64, 64),
                 F_IN, 64, stage=False)
    y2 = _tc_mid(S1, y1, deg, W1, b1r, W2p)
    S2 = _agg_sc(y2, src.reshape(E // 128, 128), dst.reshape(E // 128, 128),
                 PAD_OUT, 128, stage=True)
    return _tc_post(S2, y2, deg, b2r)
